# baseline (speedup 1.0000x reference)
import jax
import jax.numpy as jnp
import numpy as np
from jax.experimental import pallas as pl
from jax.experimental.pallas import tpu as pltpu

N_TOKENS = 64
HIDDEN = 1024
INTER = 1024
N_EXPERTS = 16
TOP_K = 2
SWIGLU_LIMIT = 7.0
EPS = 1e-05
ALPHA = 1.702

N_CHUNKS = 4
ROWS1 = 2 * INTER // N_CHUNKS
ROWS2 = INTER // N_CHUNKS
W2_SPLIT = 2


N_SLOTS = 3


def _w1_copy(w1_hbm, w1b, sem1, ei, slot, j):
    return pltpu.make_async_copy(
        w1_hbm.at[ei, pl.ds(ROWS1 * j, ROWS1), :],
        w1b.at[slot, pl.ds(ROWS1 * j, ROWS1), :],
        sem1.at[slot, j])


def _w2_copy(w2_hbm, w2b, sem2, ei, slot, j):
    rows = HIDDEN // W2_SPLIT
    return pltpu.make_async_copy(
        w2_hbm.at[ei, pl.ds(rows * j, rows), :],
        w2b.at[slot, pl.ds(rows * j, rows), :],
        sem2.at[slot, j])


def _start_fetch(w1_hbm, w2_hbm, w1b, w2b, sem1, sem2, ei, slot):
    for j in range(N_CHUNKS):
        _w1_copy(w1_hbm, w1b, sem1, ei, slot, j).start()
    for j in range(W2_SPLIT):
        _w2_copy(w2_hbm, w2b, sem2, ei, slot, j).start()


def _moe_kernel(xt_ref, scale_ref, eit_ref, elt_ref, w1_ref, b1_ref, w2_ref,
                b2_ref, pe_ref, out_ref, w1b, w2b, tn_ref, wt_ref,
                sem1, sem2):
    e = pl.program_id(0)
    slot = jax.lax.rem(e, N_SLOTS)
    sidx = jax.lax.broadcasted_iota(jnp.int32, (N_EXPERTS, N_TOKENS), 0)

    @pl.when(e == 0)
    def _prologue():
        _start_fetch(w1_ref, w2_ref, w1b, w2b, sem1, sem2, 0, 0)
        _start_fetch(w1_ref, w2_ref, w1b, w2b, sem1, sem2, 1, 1)
        xv = xt_ref[...]
        ms = jnp.mean(xv * xv, axis=0, keepdims=True)
        tn_ref[...] = ((xv / jnp.sqrt(ms + EPS))
                       * scale_ref[...]).astype(jnp.bfloat16)
        i1 = eit_ref[0:1, :]
        i2 = eit_ref[1:2, :]
        l1 = elt_ref[0:1, :]
        l2 = elt_ref[1:2, :]
        e2 = jnp.exp(l2 - l1)
        denom = 1.0 + e2
        wt_ref[...] = (jnp.where(sidx == i1, 1.0 / denom, 0.0)
                       + jnp.where(sidx == i2, e2 / denom, 0.0))
        out_ref[...] = xv

    @pl.when(e + 2 < N_EXPERTS)
    def _prefetch_next():
        _start_fetch(w1_ref, w2_ref, w1b, w2b, sem1, sem2, e + 2,
                     jax.lax.rem(e + 2, N_SLOTS))

    tn = tn_ref[...]
    pe = pe_ref[...]
    y = jnp.zeros((HIDDEN, N_TOKENS), jnp.float32)
    for j in range(W2_SPLIT):
        _w2_copy(w2_ref, w2b, sem2, e, slot, j).wait()
    for j in range(N_CHUNKS):
        _w1_copy(w1_ref, w1b, sem1, e, slot, j).wait()
        w1c = w1b[slot, pl.ds(ROWS1 * j, ROWS1), :]
        h = jax.lax.dot_general(w1c, tn, (((1,), (0,)), ((), ())),
                                preferred_element_type=jnp.float32)
        hb = (h + b1_ref[0, pl.ds(ROWS1 * j, ROWS1), :]).astype(jnp.bfloat16)
        glu = jnp.minimum(hb, SWIGLU_LIMIT)
        sg = glu * jax.nn.sigmoid(ALPHA * glu)
        lin = jnp.clip(hb, -SWIGLU_LIMIT, SWIGLU_LIMIT) + 1.0
        prod = sg * pltpu.roll(lin, ROWS1 - 1, 0)
        act = jax.lax.dot_general(pe, prod, (((1,), (0,)), ((), ())),
                                  preferred_element_type=jnp.float32
                                  ).astype(jnp.bfloat16)
        w2c = w2b[slot, :, pl.ds(ROWS2 * j, ROWS2)]
        y = y + jax.lax.dot_general(w2c, act, (((1,), (0,)), ((), ())),
                                    preferred_element_type=jnp.float32)
    y = y + b2_ref[0]
    wrow = jnp.sum(jnp.where(sidx == e, wt_ref[...], 0.0), axis=0,
                   keepdims=True)
    out_ref[...] += y * wrow


def _even_row_selector():
    pe = np.zeros((ROWS2, ROWS1), np.float32)
    pe[np.arange(ROWS2), 2 * np.arange(ROWS2)] = 1.0
    return jnp.asarray(pe, jnp.bfloat16)


@jax.jit
def kernel(x, scale, gate_kernel, gate_bias, mlp1_weight, mlp1_bias,
           mlp2_weight, mlp2_bias):
    t = x.astype(jnp.float32)
    rms = jnp.sqrt(jnp.mean(t ** 2, axis=-1, keepdims=True) + EPS)
    t = ((t / rms) * scale).astype(jnp.bfloat16)
    g = jnp.matmul(t, gate_kernel) + gate_bias
    expert_logits, expert_indices = jax.lax.top_k(g, TOP_K)

    hbm = pltpu.MemorySpace.HBM
    out_t = pl.pallas_call(
        _moe_kernel,
        grid=(N_EXPERTS,),
        in_specs=[
            pl.BlockSpec((HIDDEN, N_TOKENS), lambda e: (0, 0)),
            pl.BlockSpec((HIDDEN, 1), lambda e: (0, 0)),
            pl.BlockSpec((TOP_K, N_TOKENS), lambda e: (0, 0)),
            pl.BlockSpec((TOP_K, N_TOKENS), lambda e: (0, 0)),
            pl.BlockSpec(memory_space=hbm),
            pl.BlockSpec((1, 2 * INTER, 1), lambda e: (e, 0, 0)),
            pl.BlockSpec(memory_space=hbm),
            pl.BlockSpec((1, HIDDEN, 1), lambda e: (e, 0, 0)),
            pl.BlockSpec((ROWS2, ROWS1), lambda e: (0, 0)),
        ],
        out_specs=pl.BlockSpec((HIDDEN, N_TOKENS), lambda e: (0, 0)),
        out_shape=jax.ShapeDtypeStruct((HIDDEN, N_TOKENS), jnp.float32),
        scratch_shapes=[
            pltpu.VMEM((N_SLOTS, 2 * INTER, HIDDEN), jnp.bfloat16),
            pltpu.VMEM((N_SLOTS, HIDDEN, INTER), jnp.bfloat16),
            pltpu.VMEM((HIDDEN, N_TOKENS), jnp.bfloat16),
            pltpu.VMEM((N_EXPERTS, N_TOKENS), jnp.float32),
            pltpu.SemaphoreType.DMA((N_SLOTS, N_CHUNKS)),
            pltpu.SemaphoreType.DMA((N_SLOTS, W2_SPLIT)),
        ],
    )(
        x.T,
        scale.reshape(HIDDEN, 1),
        expert_indices.T.astype(jnp.int32),
        expert_logits.T.astype(jnp.float32),
        mlp1_weight,
        mlp1_bias.astype(jnp.float32).reshape(N_EXPERTS, 2 * INTER, 1),
        mlp2_weight,
        mlp2_bias.reshape(N_EXPERTS, HIDDEN, 1),
        _even_row_selector(),
    )
    return out_t.T


# speedup vs baseline: 1.1329x; 1.1329x over previous
import jax
import jax.numpy as jnp
import numpy as np
from jax.experimental import pallas as pl
from jax.experimental.pallas import tpu as pltpu

N_TOKENS = 64
HIDDEN = 1024
INTER = 1024
N_EXPERTS = 16
TOP_K = 2
SWIGLU_LIMIT = 7.0
EPS = 1e-05
ALPHA = 1.702

N_CHUNKS = 4
ROWS1 = 2 * INTER // N_CHUNKS
ROWS2 = INTER // N_CHUNKS
W2_SPLIT = 2


def _start_fetch(w1_hbm, w2_hbm, w1b, w2b, sem1, sem2, ei, slot):
    for j in range(N_CHUNKS):
        pltpu.make_async_copy(
            w1_hbm.at[ei, pl.ds(ROWS1 * j, ROWS1), :],
            w1b.at[slot, pl.ds(ROWS1 * j, ROWS1), :],
            sem1.at[slot, j]).start()
    for j in range(W2_SPLIT):
        pltpu.make_async_copy(
            w2_hbm.at[ei, pl.ds(HIDDEN // W2_SPLIT * j, HIDDEN // W2_SPLIT), :],
            w2b.at[slot, pl.ds(HIDDEN // W2_SPLIT * j, HIDDEN // W2_SPLIT), :],
            sem2.at[slot, j]).start()


def _wait_fetch(w1_hbm, w2_hbm, w1b, w2b, sem1, sem2, ei, slot):
    for j in range(N_CHUNKS):
        pltpu.make_async_copy(
            w1_hbm.at[ei, pl.ds(ROWS1 * j, ROWS1), :],
            w1b.at[slot, pl.ds(ROWS1 * j, ROWS1), :],
            sem1.at[slot, j]).wait()
    for j in range(W2_SPLIT):
        pltpu.make_async_copy(
            w2_hbm.at[ei, pl.ds(HIDDEN // W2_SPLIT * j, HIDDEN // W2_SPLIT), :],
            w2b.at[slot, pl.ds(HIDDEN // W2_SPLIT * j, HIDDEN // W2_SPLIT), :],
            sem2.at[slot, j]).wait()


def _moe_kernel(xt_ref, scale_ref, eit_ref, elt_ref, w1_ref, b1_ref, w2_ref,
                b2_ref, pe_ref, out_ref, w1b, w2b, tn_ref, wt_ref,
                sem1, sem2):
    e = pl.program_id(0)
    slot = jax.lax.rem(e, 2)
    sidx = jax.lax.broadcasted_iota(jnp.int32, (N_EXPERTS, N_TOKENS), 0)
    eidx = jax.lax.broadcasted_iota(jnp.int32, (N_EXPERTS, 1), 0)
    ohe = (eidx == e).astype(jnp.float32)

    @pl.when(e == 0)
    def _prologue():
        _start_fetch(w1_ref, w2_ref, w1b, w2b, sem1, sem2, 0, 0)
        xv = xt_ref[...]
        ms = jnp.mean(xv * xv, axis=0, keepdims=True)
        tn_ref[...] = ((xv / jnp.sqrt(ms + EPS))
                       * scale_ref[...]).astype(jnp.bfloat16)
        i1 = eit_ref[0:1, :]
        i2 = eit_ref[1:2, :]
        l1 = elt_ref[0:1, :]
        l2 = elt_ref[1:2, :]
        e2 = jnp.exp(l2 - l1)
        denom = 1.0 + e2
        wt_ref[...] = (jnp.where(sidx == i1, 1.0 / denom, 0.0)
                       + jnp.where(sidx == i2, e2 / denom, 0.0))
        out_ref[...] = xv

    @pl.when(e + 1 < N_EXPERTS)
    def _prefetch_next():
        _start_fetch(w1_ref, w2_ref, w1b, w2b, sem1, sem2, e + 1, 1 - slot)

    _wait_fetch(w1_ref, w2_ref, w1b, w2b, sem1, sem2, e, slot)

    tn = tn_ref[...]
    pe = pe_ref[...]
    y = jnp.zeros((HIDDEN, N_TOKENS), jnp.float32)
    for j in range(N_CHUNKS):
        w1c = w1b[slot, pl.ds(ROWS1 * j, ROWS1), :]
        h = jax.lax.dot_general(w1c, tn, (((1,), (0,)), ((), ())),
                                preferred_element_type=jnp.float32)
        b1c = jax.lax.dot_general(b1_ref[pl.ds(ROWS1 * j, ROWS1), :], ohe,
                                  (((1,), (0,)), ((), ())),
                                  preferred_element_type=jnp.float32)
        hb = (h + b1c).astype(jnp.bfloat16)
        glu = jnp.minimum(hb, SWIGLU_LIMIT)
        sg = glu * jax.nn.sigmoid(ALPHA * glu)
        lin = jnp.clip(hb, -SWIGLU_LIMIT, SWIGLU_LIMIT) + 1.0
        prod = sg * pltpu.roll(lin, ROWS1 - 1, 0)
        act = jax.lax.dot_general(pe, prod, (((1,), (0,)), ((), ())),
                                  preferred_element_type=jnp.float32
                                  ).astype(jnp.bfloat16)
        w2c = w2b[slot, :, pl.ds(ROWS2 * j, ROWS2)]
        y = y + jax.lax.dot_general(w2c, act, (((1,), (0,)), ((), ())),
                                    preferred_element_type=jnp.float32)
    y = y + jax.lax.dot_general(b2_ref[...], ohe, (((1,), (0,)), ((), ())),
                                preferred_element_type=jnp.float32)
    wrow = jnp.sum(jnp.where(sidx == e, wt_ref[...], 0.0), axis=0,
                   keepdims=True)
    out_ref[...] += y * wrow


def _even_row_selector():
    pe = np.zeros((ROWS2, ROWS1), np.float32)
    pe[np.arange(ROWS2), 2 * np.arange(ROWS2)] = 1.0
    return jnp.asarray(pe, jnp.bfloat16)


@jax.jit
def kernel(x, scale, gate_kernel, gate_bias, mlp1_weight, mlp1_bias,
           mlp2_weight, mlp2_bias):
    t = x.astype(jnp.float32)
    rms = jnp.sqrt(jnp.mean(t ** 2, axis=-1, keepdims=True) + EPS)
    t = ((t / rms) * scale).astype(jnp.bfloat16)
    g = jnp.matmul(t, gate_kernel) + gate_bias
    expert_logits, expert_indices = jax.lax.top_k(g, TOP_K)

    hbm = pltpu.MemorySpace.HBM
    out_t = pl.pallas_call(
        _moe_kernel,
        grid=(N_EXPERTS,),
        in_specs=[
            pl.BlockSpec((HIDDEN, N_TOKENS), lambda e: (0, 0)),
            pl.BlockSpec((HIDDEN, 1), lambda e: (0, 0)),
            pl.BlockSpec((TOP_K, N_TOKENS), lambda e: (0, 0)),
            pl.BlockSpec((TOP_K, N_TOKENS), lambda e: (0, 0)),
            pl.BlockSpec(memory_space=hbm),
            pl.BlockSpec((2 * INTER, N_EXPERTS), lambda e: (0, 0)),
            pl.BlockSpec(memory_space=hbm),
            pl.BlockSpec((HIDDEN, N_EXPERTS), lambda e: (0, 0)),
            pl.BlockSpec((ROWS2, ROWS1), lambda e: (0, 0)),
        ],
        out_specs=pl.BlockSpec((HIDDEN, N_TOKENS), lambda e: (0, 0)),
        out_shape=jax.ShapeDtypeStruct((HIDDEN, N_TOKENS), jnp.float32),
        scratch_shapes=[
            pltpu.VMEM((2, 2 * INTER, HIDDEN), jnp.bfloat16),
            pltpu.VMEM((2, HIDDEN, INTER), jnp.bfloat16),
            pltpu.VMEM((HIDDEN, N_TOKENS), jnp.bfloat16),
            pltpu.VMEM((N_EXPERTS, N_TOKENS), jnp.float32),
            pltpu.SemaphoreType.DMA((2, N_CHUNKS)),
            pltpu.SemaphoreType.DMA((2, W2_SPLIT)),
        ],
    )(
        x.T,
        scale.reshape(HIDDEN, 1),
        expert_indices.T.astype(jnp.int32),
        expert_logits.T.astype(jnp.float32),
        mlp1_weight,
        mlp1_bias.astype(jnp.float32).T,
        mlp2_weight,
        mlp2_bias.astype(jnp.float32).T,
        _even_row_selector(),
    )
    return out_t.T
